# baseline (speedup 1.0000x reference)
import functools

import jax
import jax.numpy as jnp
from jax import lax
from jax.experimental import pallas as pl
from jax.experimental.pallas import tpu as pltpu
from jax.experimental.pallas import tpu_sc as plsc

_EMB = 64
_LABEL = 5
_B = 4096
_L = 200
_LH = _L // 2
_NC = 2
_NS = 16
_NW = _NC * _NS
_BPW = _B // _NW
_LANES = 16
_NV = _EMB // _LANES


def _sc_sum(src3, table):
    mesh = plsc.VectorSubcoreMesh(core_axis_name="c", subcore_axis_name="s")

    @functools.partial(
        pl.kernel,
        out_type=jax.ShapeDtypeStruct((_B, _EMB), jnp.float32),
        mesh=mesh,
        scratch_types=[
            pltpu.VMEM((_BPW, 2, _LH), jnp.int32),
            pltpu.VMEM((_L, _EMB), jnp.float32),
            pltpu.VMEM((_BPW, _EMB), jnp.float32),
            pltpu.SemaphoreType.DMA,
        ],
        compiler_params=pltpu.CompilerParams(use_tc_tiling_on_sc=False),
    )
    def k(src_hbm, table_hbm, out_hbm, idx_v, rows_v, out_v, sem):
        wid = lax.axis_index("s") * _NC + lax.axis_index("c")
        base = wid * _BPW
        pltpu.sync_copy(src_hbm.at[pl.ds(base, _BPW)], idx_v)

        def body(i, _):
            d0 = pltpu.async_copy(
                table_hbm.at[idx_v.at[i, 0]], rows_v.at[pl.ds(0, _LH)], sem)
            d1 = pltpu.async_copy(
                table_hbm.at[idx_v.at[i, 1]], rows_v.at[pl.ds(_LH, _LH)], sem)
            d0.wait()
            d1.wait()

            zero = jnp.zeros((_LANES,), jnp.float32)
            accs = [zero] * (2 * _NV)

            def red(lb, accs):
                accs = list(accs)
                for j in range(8):
                    l = lb * 8 + j
                    p = (j % 2) * _NV
                    for d in range(_NV):
                        accs[p + d] = accs[p + d] + rows_v[
                            l, pl.ds(d * _LANES, _LANES)]
                return tuple(accs)

            accs = lax.fori_loop(0, _L // 8, red, tuple(accs))
            for d in range(_NV):
                out_v[i, pl.ds(d * _LANES, _LANES)] = accs[d] + accs[_NV + d]
            return 0

        lax.fori_loop(0, _BPW, body, 0)
        pltpu.sync_copy(out_v, out_hbm.at[pl.ds(base, _BPW)])

    return k(src3, table)


def _tc_head(x_ref, w_ref, b_ref, o_ref):
    x = x_ref[...] * (1.0 / _L)
    logits = lax.dot_general(
        x, w_ref[...], (((1,), (1,)), ((), ())),
        preferred_element_type=jnp.float32) + b_ref[...]
    m = jnp.max(logits, axis=1, keepdims=True)
    s = logits - m
    o_ref[...] = s - jnp.log(jnp.sum(jnp.exp(s), axis=1, keepdims=True))


def kernel(src, src_lengths, emb_table, W, b):
    del src_lengths
    src3 = src.astype(jnp.int32).reshape(_B, 2, _LH)
    sums = _sc_sum(src3, emb_table)
    return pl.pallas_call(
        _tc_head,
        out_shape=jax.ShapeDtypeStruct((_B, _LABEL), jnp.float32),
    )(sums, W, b.reshape(1, _LABEL))


# speedup vs baseline: 1.1907x; 1.1907x over previous
import functools

import jax
import jax.numpy as jnp
from jax import lax
from jax.experimental import pallas as pl
from jax.experimental.pallas import tpu as pltpu
from jax.experimental.pallas import tpu_sc as plsc

_EMB = 64
_LABEL = 5
_B = 4096
_L = 200
_LH = _L // 2
_NC = 2
_NS = 16
_NW = _NC * _NS
_BPW = _B // _NW
_LANES = 16
_NV = _EMB // _LANES
_NBUF = 4


def _sc_sum(src3, table):
    mesh = plsc.VectorSubcoreMesh(core_axis_name="c", subcore_axis_name="s")

    @functools.partial(
        pl.kernel,
        out_type=jax.ShapeDtypeStruct((_B, _EMB), jnp.float32),
        mesh=mesh,
        scratch_types=[
            pltpu.VMEM((_BPW, 2, _LH), jnp.int32),
            pltpu.VMEM((_NBUF, _L, _EMB), jnp.float32),
            pltpu.VMEM((_BPW, _EMB), jnp.float32),
            pltpu.SemaphoreType.DMA,
            pltpu.SemaphoreType.DMA,
            pltpu.SemaphoreType.DMA,
            pltpu.SemaphoreType.DMA,
        ],
        compiler_params=pltpu.CompilerParams(use_tc_tiling_on_sc=False),
    )
    def k(src_hbm, table_hbm, out_hbm, idx_v, rows_v, out_v, s0, s1, s2, s3):
        sems = [s0, s1, s2, s3]
        wid = lax.axis_index("s") * _NC + lax.axis_index("c")
        base = wid * _BPW
        pltpu.sync_copy(src_hbm.at[pl.ds(base, _BPW)], idx_v)

        def issue(i, j):
            pltpu.async_copy(
                table_hbm.at[idx_v.at[i, 0]], rows_v.at[j, pl.ds(0, _LH)],
                sems[j])
            pltpu.async_copy(
                table_hbm.at[idx_v.at[i, 1]], rows_v.at[j, pl.ds(_LH, _LH)],
                sems[j])

        for j in range(_NBUF):
            issue(j, j)

        def group(g, _):
            for j in range(_NBUF):
                i = g * _NBUF + j
                pltpu.make_async_copy(
                    table_hbm.at[pl.ds(0, _L)], rows_v.at[j], sems[j]).wait()

                zero = jnp.zeros((_LANES,), jnp.float32)
                accs = [zero] * (2 * _NV)

                def red(lb, accs, j=j):
                    accs = list(accs)
                    for u in range(8):
                        l = lb * 8 + u
                        p = (u % 2) * _NV
                        for d in range(_NV):
                            accs[p + d] = accs[p + d] + rows_v[
                                j, l, pl.ds(d * _LANES, _LANES)]
                    return tuple(accs)

                accs = lax.fori_loop(0, _L // 8, red, tuple(accs))
                for d in range(_NV):
                    out_v[i, pl.ds(d * _LANES, _LANES)] = (
                        accs[d] + accs[_NV + d])

                @pl.when(i + _NBUF < _BPW)
                def _(i=i, j=j):
                    issue(i + _NBUF, j)
            return 0

        lax.fori_loop(0, _BPW // _NBUF, group, 0)
        pltpu.sync_copy(out_v, out_hbm.at[pl.ds(base, _BPW)])

    return k(src3, table)


def _tc_head(x_ref, w_ref, b_ref, o_ref):
    x = x_ref[...] * (1.0 / _L)
    logits = lax.dot_general(
        x, w_ref[...], (((1,), (1,)), ((), ())),
        preferred_element_type=jnp.float32) + b_ref[...]
    m = jnp.max(logits, axis=1, keepdims=True)
    s = logits - m
    o_ref[...] = s - jnp.log(jnp.sum(jnp.exp(s), axis=1, keepdims=True))


def kernel(src, src_lengths, emb_table, W, b):
    del src_lengths
    src3 = src.astype(jnp.int32).reshape(_B, 2, _LH)
    sums = _sc_sum(src3, emb_table)
    return pl.pallas_call(
        _tc_head,
        out_shape=jax.ShapeDtypeStruct((_B, _LABEL), jnp.float32),
    )(sums, W, b.reshape(1, _LABEL))
